# Initial kernel scaffold; baseline (speedup 1.0000x reference)
import jax
import jax.numpy as jnp
from jax.experimental import pallas as pl


def kernel(x, edges, W, b, W2, b2):
    raise NotImplementedError("write your pallas kernel here")


# speedup vs baseline: 88.0660x; 88.0660x over previous
import functools

import jax
import jax.numpy as jnp
from jax import lax
from jax.experimental import pallas as pl
from jax.experimental.pallas import tpu as pltpu
from jax.experimental.pallas import tpu_sc as plsc

N_NODES = 10000
N_EDGES = 320000
D_IN = 128
D_HID = 3
D_OUT = 4

_NC, _NS, _L = 2, 16, 16
_NW = _NC * _NS
_EPW = N_EDGES // _NW


@functools.cache
def _sc_kernels():
    mesh = plsc.VectorSubcoreMesh(
        core_axis_name="c", subcore_axis_name="s",
        num_cores=_NC, num_subcores=_NS)
    params = pltpu.CompilerParams(needs_layout_passes=False)

    @functools.partial(
        pl.kernel,
        mesh=mesh,
        out_type=jax.ShapeDtypeStruct((_NW, N_NODES), jnp.float32),
        compiler_params=params,
        scratch_types=[
            pltpu.VMEM((_EPW,), jnp.int32),
            pltpu.VMEM((N_NODES,), jnp.float32),
        ],
    )
    def sc_hist(dst_hbm, out_hbm, dst_v, deg_v):
        wid = lax.axis_index("s") * _NC + lax.axis_index("c")
        pltpu.sync_copy(dst_hbm.at[pl.ds(wid * _EPW, _EPW)], dst_v)

        zeros = jnp.zeros((_L,), jnp.float32)

        def zbody(i, c):
            deg_v[pl.ds(i * _L, _L)] = zeros
            return c

        lax.fori_loop(0, N_NODES // _L, zbody, 0)

        ones = jnp.ones((_L,), jnp.float32)

        def body(g, c):
            d16 = dst_v[pl.ds(g * _L, _L)]
            plsc.addupdate_scatter(deg_v, [d16], ones)
            return c

        lax.fori_loop(0, _EPW // _L, body, 0)
        pltpu.sync_copy(deg_v, out_hbm.at[wid])

    @functools.partial(
        pl.kernel,
        mesh=mesh,
        out_type=jax.ShapeDtypeStruct((_NW, D_HID * N_NODES), jnp.float32),
        compiler_params=params,
        scratch_types=[
            pltpu.VMEM((D_HID * N_NODES,), jnp.float32),
            pltpu.VMEM((_EPW,), jnp.int32),
            pltpu.VMEM((_EPW,), jnp.int32),
            pltpu.VMEM((D_HID * N_NODES,), jnp.float32),
        ],
    )
    def sc_edges(y_hbm, src_hbm, dst_hbm, out_hbm, y_v, src_v, dst_v, acc_v):
        wid = lax.axis_index("s") * _NC + lax.axis_index("c")
        pltpu.sync_copy(y_hbm, y_v)
        pltpu.sync_copy(src_hbm.at[pl.ds(wid * _EPW, _EPW)], src_v)
        pltpu.sync_copy(dst_hbm.at[pl.ds(wid * _EPW, _EPW)], dst_v)

        zeros = jnp.zeros((_L,), jnp.float32)

        def zbody(i, c):
            acc_v[pl.ds(i * _L, _L)] = zeros
            return c

        lax.fori_loop(0, D_HID * N_NODES // _L, zbody, 0)

        def body(g, c):
            base = g * _L
            s16 = src_v[pl.ds(base, _L)]
            d16 = dst_v[pl.ds(base, _L)]
            for ch in range(D_HID):
                off = jnp.int32(ch * N_NODES)
                v = plsc.load_gather(y_v, [s16 + off])
                plsc.addupdate_scatter(acc_v, [d16 + off], v)
            return c

        lax.fori_loop(0, _EPW // _L, body, 0)
        pltpu.sync_copy(acc_v, out_hbm.at[wid])

    return sc_hist, sc_edges


def _mid_body(x_ref, w_ref, degp_ref, y_ref, dinv_ref):
    xw_t = lax.dot_general(
        w_ref[...], x_ref[...], (((0,), (1,)), ((), ())),
        preferred_element_type=jnp.float32)
    deg = jnp.sum(degp_ref[...], axis=0, keepdims=True) + 1.0
    dinv = lax.rsqrt(deg)
    dinv_ref[...] = dinv
    y_ref[...] = xw_t * dinv


def _fin_body(accp_ref, y_ref, dinv_ref, b_ref, w2_ref, b2_ref, h_ref, z_ref):
    acc = jnp.sum(accp_ref[...], axis=0)
    h = jnp.maximum(dinv_ref[...] * (acc + y_ref[...]) + b_ref[...], 0.0)
    h_ref[...] = h
    z_ref[...] = lax.dot_general(
        w2_ref[...], h, (((0,), (0,)), ((), ())),
        preferred_element_type=jnp.float32) + b2_ref[...]


_mid_call = pl.pallas_call(
    _mid_body,
    out_shape=[
        jax.ShapeDtypeStruct((D_HID, N_NODES), jnp.float32),
        jax.ShapeDtypeStruct((1, N_NODES), jnp.float32),
    ],
)

_fin_call = pl.pallas_call(
    _fin_body,
    out_shape=[
        jax.ShapeDtypeStruct((D_HID, N_NODES), jnp.float32),
        jax.ShapeDtypeStruct((D_OUT, N_NODES), jnp.float32),
    ],
)


def kernel(x, edges, W, b, W2, b2):
    src = edges[0].astype(jnp.int32)
    dst = edges[1].astype(jnp.int32)

    sc_hist, sc_edges = _sc_kernels()
    degp = sc_hist(dst)
    y_t, dinv = _mid_call(x, W, degp)
    accp = sc_edges(y_t.reshape(D_HID * N_NODES), src, dst)
    h_t, z_t = _fin_call(
        accp.reshape(_NW, D_HID, N_NODES), y_t, dinv,
        b.reshape(D_HID, 1), W2, b2.reshape(D_OUT, 1))
    return h_t.T, z_t.T


# speedup vs baseline: 97.4580x; 1.1066x over previous
import functools

import jax
import jax.numpy as jnp
from jax import lax
from jax.experimental import pallas as pl
from jax.experimental.pallas import tpu as pltpu
from jax.experimental.pallas import tpu_sc as plsc

N_NODES = 10000
N_EDGES = 320000
D_IN = 128
D_HID = 3
D_OUT = 4

_NC, _NS, _L = 2, 16, 16
_NW = _NC * _NS
_EPW = N_EDGES // _NW


@functools.cache
def _sc_kernels():
    mesh = plsc.VectorSubcoreMesh(
        core_axis_name="c", subcore_axis_name="s",
        num_cores=_NC, num_subcores=_NS)
    params = pltpu.CompilerParams(needs_layout_passes=False)

    @functools.partial(
        pl.kernel,
        mesh=mesh,
        out_type=jax.ShapeDtypeStruct((_NW, N_NODES), jnp.float32),
        compiler_params=params,
        scratch_types=[
            pltpu.VMEM((_EPW,), jnp.int32),
            pltpu.VMEM((N_NODES,), jnp.float32),
        ],
    )
    def sc_hist(dst_hbm, out_hbm, dst_v, deg_v):
        wid = lax.axis_index("s") * _NC + lax.axis_index("c")
        pltpu.sync_copy(dst_hbm.at[pl.ds(wid * _EPW, _EPW)], dst_v)

        zeros = jnp.zeros((_L,), jnp.float32)

        def zbody(i, c):
            deg_v[pl.ds(i * _L, _L)] = zeros
            return c

        lax.fori_loop(0, N_NODES // _L, zbody, 0, unroll=8)

        ones = jnp.ones((_L,), jnp.float32)

        def body(g, c):
            d16 = dst_v[pl.ds(g * _L, _L)]
            plsc.addupdate_scatter(deg_v, [d16], ones)
            return c

        lax.fori_loop(0, _EPW // _L, body, 0, unroll=8)
        pltpu.sync_copy(deg_v, out_hbm.at[wid])

    @functools.partial(
        pl.kernel,
        mesh=mesh,
        out_type=jax.ShapeDtypeStruct((_NW, D_HID * N_NODES), jnp.float32),
        compiler_params=params,
        scratch_types=[
            pltpu.VMEM((D_HID * N_NODES,), jnp.float32),
            pltpu.VMEM((_EPW,), jnp.int32),
            pltpu.VMEM((_EPW,), jnp.int32),
            pltpu.VMEM((D_HID * N_NODES,), jnp.float32),
        ],
    )
    def sc_edges(y_hbm, src_hbm, dst_hbm, out_hbm, y_v, src_v, dst_v, acc_v):
        wid = lax.axis_index("s") * _NC + lax.axis_index("c")
        pltpu.sync_copy(y_hbm, y_v)
        pltpu.sync_copy(src_hbm.at[pl.ds(wid * _EPW, _EPW)], src_v)
        pltpu.sync_copy(dst_hbm.at[pl.ds(wid * _EPW, _EPW)], dst_v)

        zeros = jnp.zeros((_L,), jnp.float32)

        def zbody(i, c):
            acc_v[pl.ds(i * _L, _L)] = zeros
            return c

        lax.fori_loop(0, D_HID * N_NODES // _L, zbody, 0, unroll=8)

        def body(g, c):
            base = g * _L
            s16 = src_v[pl.ds(base, _L)]
            d16 = dst_v[pl.ds(base, _L)]
            for ch in range(D_HID):
                off = jnp.int32(ch * N_NODES)
                v = plsc.load_gather(y_v, [s16 + off])
                plsc.addupdate_scatter(acc_v, [d16 + off], v)
            return c

        lax.fori_loop(0, _EPW // _L, body, 0, unroll=4)
        pltpu.sync_copy(acc_v, out_hbm.at[wid])

    return sc_hist, sc_edges


def _mid_body(x_ref, w_ref, degp_ref, y_ref, dinv_ref):
    xw_t = lax.dot_general(
        w_ref[...], x_ref[...], (((0,), (1,)), ((), ())),
        preferred_element_type=jnp.float32)
    deg = jnp.sum(degp_ref[...], axis=0, keepdims=True) + 1.0
    dinv = lax.rsqrt(deg)
    dinv_ref[...] = dinv
    y_ref[...] = xw_t * dinv


def _fin_body(accp_ref, y_ref, dinv_ref, b_ref, w2_ref, b2_ref, h_ref, z_ref):
    acc = jnp.sum(accp_ref[...], axis=0)
    h = jnp.maximum(dinv_ref[...] * (acc + y_ref[...]) + b_ref[...], 0.0)
    h_ref[...] = h
    z_ref[...] = lax.dot_general(
        w2_ref[...], h, (((0,), (0,)), ((), ())),
        preferred_element_type=jnp.float32) + b2_ref[...]


_mid_call = pl.pallas_call(
    _mid_body,
    out_shape=[
        jax.ShapeDtypeStruct((D_HID, N_NODES), jnp.float32),
        jax.ShapeDtypeStruct((1, N_NODES), jnp.float32),
    ],
)

_fin_call = pl.pallas_call(
    _fin_body,
    out_shape=[
        jax.ShapeDtypeStruct((D_HID, N_NODES), jnp.float32),
        jax.ShapeDtypeStruct((D_OUT, N_NODES), jnp.float32),
    ],
)


def kernel(x, edges, W, b, W2, b2):
    src = edges[0].astype(jnp.int32)
    dst = edges[1].astype(jnp.int32)

    sc_hist, sc_edges = _sc_kernels()
    degp = sc_hist(dst)
    y_t, dinv = _mid_call(x, W, degp)
    accp = sc_edges(y_t.reshape(D_HID * N_NODES), src, dst)
    h_t, z_t = _fin_call(
        accp.reshape(_NW, D_HID, N_NODES), y_t, dinv,
        b.reshape(D_HID, 1), W2, b2.reshape(D_OUT, 1))
    return h_t.T, z_t.T


# speedup vs baseline: 111.7051x; 1.1462x over previous
import functools

import jax
import jax.numpy as jnp
from jax import lax
from jax.experimental import pallas as pl
from jax.experimental.pallas import tpu as pltpu
from jax.experimental.pallas import tpu_sc as plsc

N_NODES = 10000
N_EDGES = 320000
D_IN = 128
D_HID = 3
D_OUT = 4

_NC, _NS, _L = 2, 16, 16
_NW = _NC * _NS
_EPW = N_EDGES // _NW


@functools.cache
def _sc_kernels():
    mesh = plsc.VectorSubcoreMesh(
        core_axis_name="c", subcore_axis_name="s",
        num_cores=_NC, num_subcores=_NS)
    params = pltpu.CompilerParams(needs_layout_passes=False)

    @functools.partial(
        pl.kernel,
        mesh=mesh,
        out_type=jax.ShapeDtypeStruct((_NW, N_NODES), jnp.float32),
        compiler_params=params,
        scratch_types=[
            pltpu.VMEM((_EPW,), jnp.int32),
            pltpu.VMEM((N_NODES,), jnp.float32),
        ],
    )
    def sc_hist(dst_hbm, out_hbm, dst_v, deg_v):
        wid = lax.axis_index("s") * _NC + lax.axis_index("c")
        pltpu.sync_copy(dst_hbm.at[pl.ds(wid * _EPW, _EPW)], dst_v)

        zeros = jnp.zeros((_L,), jnp.float32)

        @plsc.parallel_loop(0, N_NODES // _L, unroll=8)
        def zbody(i):
            deg_v[pl.ds(i * _L, _L)] = zeros

        ones = jnp.ones((_L,), jnp.float32)

        @plsc.parallel_loop(0, _EPW // _L, unroll=8)
        def body(g):
            d16 = dst_v[pl.ds(g * _L, _L)]
            plsc.addupdate_scatter(deg_v, [d16], ones)
        pltpu.sync_copy(deg_v, out_hbm.at[wid])

    @functools.partial(
        pl.kernel,
        mesh=mesh,
        out_type=jax.ShapeDtypeStruct((_NW, D_HID * N_NODES), jnp.float32),
        compiler_params=params,
        scratch_types=[
            pltpu.VMEM((D_HID * N_NODES,), jnp.float32),
            pltpu.VMEM((_EPW,), jnp.int32),
            pltpu.VMEM((_EPW,), jnp.int32),
            pltpu.VMEM((D_HID * N_NODES,), jnp.float32),
        ],
    )
    def sc_edges(y_hbm, src_hbm, dst_hbm, out_hbm, y_v, src_v, dst_v, acc_v):
        wid = lax.axis_index("s") * _NC + lax.axis_index("c")
        pltpu.sync_copy(y_hbm, y_v)
        pltpu.sync_copy(src_hbm.at[pl.ds(wid * _EPW, _EPW)], src_v)
        pltpu.sync_copy(dst_hbm.at[pl.ds(wid * _EPW, _EPW)], dst_v)

        zeros = jnp.zeros((_L,), jnp.float32)

        @plsc.parallel_loop(0, D_HID * N_NODES // _L, unroll=8)
        def zbody(i):
            acc_v[pl.ds(i * _L, _L)] = zeros

        @plsc.parallel_loop(0, _EPW // _L, unroll=4)
        def body(g):
            base = g * _L
            s16 = src_v[pl.ds(base, _L)]
            d16 = dst_v[pl.ds(base, _L)]
            for ch in range(D_HID):
                off = jnp.int32(ch * N_NODES)
                v = plsc.load_gather(y_v, [s16 + off])
                plsc.addupdate_scatter(acc_v, [d16 + off], v)
        pltpu.sync_copy(acc_v, out_hbm.at[wid])

    return sc_hist, sc_edges


def _mid_body(x_ref, w_ref, degp_ref, y_ref, dinv_ref):
    xw_t = lax.dot_general(
        w_ref[...], x_ref[...], (((0,), (1,)), ((), ())),
        preferred_element_type=jnp.float32)
    deg = jnp.sum(degp_ref[...], axis=0, keepdims=True) + 1.0
    dinv = lax.rsqrt(deg)
    dinv_ref[...] = dinv
    y_ref[...] = xw_t * dinv


def _fin_body(accp_ref, y_ref, dinv_ref, b_ref, w2_ref, b2_ref, h_ref, z_ref):
    acc = jnp.sum(accp_ref[...], axis=0)
    h = jnp.maximum(dinv_ref[...] * (acc + y_ref[...]) + b_ref[...], 0.0)
    h_ref[...] = h
    z_ref[...] = lax.dot_general(
        w2_ref[...], h, (((0,), (0,)), ((), ())),
        preferred_element_type=jnp.float32) + b2_ref[...]


_mid_call = pl.pallas_call(
    _mid_body,
    out_shape=[
        jax.ShapeDtypeStruct((D_HID, N_NODES), jnp.float32),
        jax.ShapeDtypeStruct((1, N_NODES), jnp.float32),
    ],
)

_fin_call = pl.pallas_call(
    _fin_body,
    out_shape=[
        jax.ShapeDtypeStruct((D_HID, N_NODES), jnp.float32),
        jax.ShapeDtypeStruct((D_OUT, N_NODES), jnp.float32),
    ],
)


def kernel(x, edges, W, b, W2, b2):
    src = edges[0].astype(jnp.int32)
    dst = edges[1].astype(jnp.int32)

    sc_hist, sc_edges = _sc_kernels()
    degp = sc_hist(dst)
    y_t, dinv = _mid_call(x, W, degp)
    accp = sc_edges(y_t.reshape(D_HID * N_NODES), src, dst)
    h_t, z_t = _fin_call(
        accp.reshape(_NW, D_HID, N_NODES), y_t, dinv,
        b.reshape(D_HID, 1), W2, b2.reshape(D_OUT, 1))
    return h_t.T, z_t.T


# speedup vs baseline: 114.6381x; 1.0263x over previous
import functools

import jax
import jax.numpy as jnp
from jax import lax
from jax.experimental import pallas as pl
from jax.experimental.pallas import tpu as pltpu
from jax.experimental.pallas import tpu_sc as plsc

N_NODES = 10000
N_EDGES = 320000
D_IN = 128
D_HID = 3
D_OUT = 4

_NC, _NS, _L = 2, 16, 16
_NW = _NC * _NS
_EPW = N_EDGES // _NW


@functools.cache
def _sc_kernels():
    mesh = plsc.VectorSubcoreMesh(
        core_axis_name="c", subcore_axis_name="s",
        num_cores=_NC, num_subcores=_NS)
    params = pltpu.CompilerParams(needs_layout_passes=False)

    @functools.partial(
        pl.kernel,
        mesh=mesh,
        out_type=jax.ShapeDtypeStruct((_NW, N_NODES), jnp.float32),
        compiler_params=params,
        scratch_types=[
            pltpu.VMEM((_EPW,), jnp.int32),
            pltpu.VMEM((N_NODES,), jnp.float32),
        ],
    )
    def sc_hist(epk_hbm, out_hbm, epk_v, deg_v):
        wid = lax.axis_index("s") * _NC + lax.axis_index("c")
        pltpu.sync_copy(epk_hbm.at[pl.ds(wid * _EPW, _EPW)], epk_v)

        zeros = jnp.zeros((_L,), jnp.float32)

        @plsc.parallel_loop(0, N_NODES // _L, unroll=8)
        def zbody(i):
            deg_v[pl.ds(i * _L, _L)] = zeros

        ones = jnp.ones((_L,), jnp.float32)

        @plsc.parallel_loop(0, _EPW // _L, unroll=8)
        def body(g):
            d16 = lax.shift_right_logical(epk_v[pl.ds(g * _L, _L)], 14)
            plsc.addupdate_scatter(deg_v, [d16], ones)
        pltpu.sync_copy(deg_v, out_hbm.at[wid])

    @functools.partial(
        pl.kernel,
        mesh=mesh,
        out_type=jax.ShapeDtypeStruct((_NW, D_HID * N_NODES), jnp.float32),
        compiler_params=params,
        scratch_types=[
            pltpu.VMEM((D_HID * N_NODES,), jnp.float32),
            pltpu.VMEM((_EPW,), jnp.int32),
            pltpu.VMEM((D_HID * N_NODES,), jnp.float32),
        ],
    )
    def sc_edges(y_hbm, epk_hbm, out_hbm, y_v, epk_v, acc_v):
        wid = lax.axis_index("s") * _NC + lax.axis_index("c")
        pltpu.sync_copy(y_hbm, y_v)
        pltpu.sync_copy(epk_hbm.at[pl.ds(wid * _EPW, _EPW)], epk_v)

        zeros = jnp.zeros((_L,), jnp.float32)

        @plsc.parallel_loop(0, D_HID * N_NODES // _L, unroll=8)
        def zbody(i):
            acc_v[pl.ds(i * _L, _L)] = zeros

        srcmask = jnp.full((_L,), (1 << 14) - 1, jnp.int32)

        @plsc.parallel_loop(0, _EPW // _L, unroll=8)
        def body(g):
            e16 = epk_v[pl.ds(g * _L, _L)]
            s16 = e16 & srcmask
            d16 = lax.shift_right_logical(e16, 14)
            for ch in range(D_HID):
                off = jnp.int32(ch * N_NODES)
                v = plsc.load_gather(y_v, [s16 + off])
                plsc.addupdate_scatter(acc_v, [d16 + off], v)
        pltpu.sync_copy(acc_v, out_hbm.at[wid])

    return sc_hist, sc_edges


def _mid_body(x_ref, w_ref, degp_ref, y_ref, dinv_ref):
    xw_t = lax.dot_general(
        w_ref[...], x_ref[...], (((0,), (1,)), ((), ())),
        preferred_element_type=jnp.float32)
    deg = jnp.sum(degp_ref[...], axis=0, keepdims=True) + 1.0
    dinv = lax.rsqrt(deg)
    dinv_ref[...] = dinv
    y_ref[...] = xw_t * dinv


def _fin_body(accp_ref, y_ref, dinv_ref, b_ref, w2_ref, b2_ref, h_ref, z_ref):
    acc = jnp.sum(accp_ref[...], axis=0)
    h = jnp.maximum(dinv_ref[...] * (acc + y_ref[...]) + b_ref[...], 0.0)
    h_ref[...] = h
    z_ref[...] = lax.dot_general(
        w2_ref[...], h, (((0,), (0,)), ((), ())),
        preferred_element_type=jnp.float32) + b2_ref[...]


_mid_call = pl.pallas_call(
    _mid_body,
    out_shape=[
        jax.ShapeDtypeStruct((D_HID, N_NODES), jnp.float32),
        jax.ShapeDtypeStruct((1, N_NODES), jnp.float32),
    ],
)

_fin_call = pl.pallas_call(
    _fin_body,
    out_shape=[
        jax.ShapeDtypeStruct((D_HID, N_NODES), jnp.float32),
        jax.ShapeDtypeStruct((D_OUT, N_NODES), jnp.float32),
    ],
)


def kernel(x, edges, W, b, W2, b2):
    src = edges[0].astype(jnp.int32)
    dst = edges[1].astype(jnp.int32)
    packed = src | (dst << jnp.int32(14))

    sc_hist, sc_edges = _sc_kernels()
    degp = sc_hist(packed)
    y_t, dinv = _mid_call(x, W, degp)
    accp = sc_edges(y_t.reshape(D_HID * N_NODES), packed)
    h_t, z_t = _fin_call(
        accp.reshape(_NW, D_HID, N_NODES), y_t, dinv,
        b.reshape(D_HID, 1), W2, b2.reshape(D_OUT, 1))
    return h_t.T, z_t.T
